# baseline (speedup 1.0000x reference)
import functools

import jax
import jax.numpy as jnp
from jax import lax
from jax.experimental import pallas as pl
from jax.experimental.pallas import tpu as pltpu
from jax.experimental.pallas import tpu_sc as plsc

N_NODES = 10000
N_EDGES = 160000
D = 128
N_BLOCK = 4

_NC = 2
_NS = 16
_NW = _NC * _NS

E_H1 = 81920
E_H2 = N_EDGES - E_H1

ZC = 80
NZ = N_NODES // ZC
ZPW = -(-NZ // _NS)

_mesh = plsc.VectorSubcoreMesh(core_axis_name="c", subcore_axis_name="s")

GE = 80


def _make_gather(eoff, ecnt):
    ept = ecnt // _NS
    gn = ept // GE
    assert ept * _NS == ecnt and gn * GE == ept and ept % 8 == 0

    @functools.partial(
        pl.kernel,
        out_type=(jax.ShapeDtypeStruct((ecnt, D), jnp.float32),
                  jax.ShapeDtypeStruct((ecnt, D), jnp.float32)),
        mesh=_mesh,
        scratch_types=[
            pltpu.VMEM_SHARED((N_NODES, D), jnp.float32),
            pltpu.VMEM((GE,), jnp.int32),
            pltpu.VMEM((GE,), jnp.int32),
            pltpu.VMEM((GE, D), jnp.float32),
            pltpu.VMEM((GE, D), jnp.float32),
            pltpu.SemaphoreType.DMA,
            pltpu.SemaphoreType.DMA,
            pltpu.SemaphoreType.DMA,
            pltpu.SemaphoreType.DMA,
            pltpu.SemaphoreType.DMA,
            pltpu.SemaphoreType.DMA,
        ],
    )
    def gather(p_hbm, q_hbm, src_hbm, dst_hbm, gs_hbm, gd_hbm,
               tbl_sh, idx0, idx1, buf0, buf1,
               isem0, isem1, gsem0, gsem1, wsem0, wsem1):
        cid = lax.axis_index("c")
        sid = lax.axis_index("s")
        idxs = (idx0, idx1)
        bufs = (buf0, buf1)
        isems = (isem0, isem1)
        gsems = (gsem0, gsem1)
        wsems = (wsem0, wsem1)

        def stage(tab_hbm):
            def st(z, carry):
                ci = z * _NS + sid

                @pl.when(ci < NZ)
                def _():
                    base = pl.multiple_of(ci * ZC, 8)
                    pltpu.sync_copy(tab_hbm.at[pl.ds(base, ZC)],
                                    tbl_sh.at[pl.ds(base, ZC)])

                return carry

            lax.fori_loop(0, ZPW, st, 0)

        @pl.when(cid == 0)
        def _():
            stage(p_hbm)

        @pl.when(cid == 1)
        def _():
            stage(q_hbm)

        plsc.subcore_barrier()

        def run(ind_hbm, out_hbm):
            tb_i = eoff + sid * ept
            tb_o = sid * ept

            def idxload(k, b):
                pltpu.async_copy(ind_hbm.at[pl.ds(tb_i + k * GE, GE)],
                                 idxs[b], isems[b])

            idxload(0, 0)
            idxload(1, 1)
            pltpu.make_async_copy(ind_hbm.at[pl.ds(0, GE)], idx0,
                                  isem0).wait()
            pltpu.async_copy(tbl_sh.at[idx0], buf0, gsem0)

            def step(k, b):
                o = 1 - b
                pltpu.make_async_copy(tbl_sh.at[idxs[b]], bufs[b],
                                      gsems[b]).wait()

                @pl.when(k + 2 < gn)
                def _():
                    idxload(k + 2, b)

                pltpu.async_copy(bufs[b],
                                 out_hbm.at[pl.ds(tb_o + k * GE, GE)],
                                 wsems[b])
                pltpu.make_async_copy(ind_hbm.at[pl.ds(0, GE)], idxs[o],
                                      isems[o]).wait()

                @pl.when(k >= 1)
                def _():
                    pltpu.make_async_copy(bufs[o],
                                          out_hbm.at[pl.ds(0, GE)],
                                          wsems[o]).wait()

                pltpu.async_copy(tbl_sh.at[idxs[o]], bufs[o], gsems[o])

            def body(k2, carry):
                step(k2 * 2, 0)
                step(k2 * 2 + 1, 1)
                return carry

            lax.fori_loop(0, (gn - 1) // 2, body, 0)

            if gn % 2 == 1:
                pltpu.make_async_copy(tbl_sh.at[idx0], buf0,
                                      gsem0).wait()
                pltpu.async_copy(
                    buf0, out_hbm.at[pl.ds(tb_o + (gn - 1) * GE, GE)],
                    wsem0)
                pltpu.make_async_copy(buf1, out_hbm.at[pl.ds(0, GE)],
                                      wsem1).wait()
                pltpu.make_async_copy(buf0, out_hbm.at[pl.ds(0, GE)],
                                      wsem0).wait()
            else:
                pltpu.make_async_copy(tbl_sh.at[idx0], buf0,
                                      gsem0).wait()
                pltpu.async_copy(
                    buf0, out_hbm.at[pl.ds(tb_o + (gn - 2) * GE, GE)],
                    wsem0)
                pltpu.make_async_copy(ind_hbm.at[pl.ds(0, GE)], idx1,
                                      isem1).wait()
                pltpu.make_async_copy(buf1, out_hbm.at[pl.ds(0, GE)],
                                      wsem1).wait()
                pltpu.async_copy(tbl_sh.at[idx1], buf1, gsem1)
                pltpu.make_async_copy(tbl_sh.at[idx1], buf1,
                                      gsem1).wait()
                pltpu.async_copy(
                    buf1, out_hbm.at[pl.ds(tb_o + (gn - 1) * GE, GE)],
                    wsem1)
                pltpu.make_async_copy(buf0, out_hbm.at[pl.ds(0, GE)],
                                      wsem0).wait()
                pltpu.make_async_copy(buf1, out_hbm.at[pl.ds(0, GE)],
                                      wsem1).wait()

        @pl.when(cid == 0)
        def _():
            run(src_hbm, gs_hbm)

        @pl.when(cid == 1)
        def _():
            run(dst_hbm, gd_hbm)

    return gather


def _make_scatter(eoff, ecnt, se):
    epw = ecnt // _NW
    sn = epw // se
    assert epw * _NW == ecnt and sn * se == epw and epw % 8 == 0
    assert sn >= 2 and se % 8 == 0

    @functools.partial(
        pl.kernel,
        out_type=jax.ShapeDtypeStruct((_NC, N_NODES, D), jnp.float32),
        mesh=_mesh,
        scratch_types=[
            pltpu.VMEM_SHARED((N_NODES, D), jnp.float32),
            pltpu.VMEM((se, D), jnp.float32),
            pltpu.VMEM((se, D), jnp.float32),
            pltpu.VMEM((se,), jnp.int32),
            pltpu.VMEM((se,), jnp.int32),
            pltpu.SemaphoreType.DMA,
            pltpu.SemaphoreType.DMA,
            pltpu.SemaphoreType.DMA,
            pltpu.SemaphoreType.DMA,
            pltpu.SemaphoreType.DMA,
            pltpu.SemaphoreType.DMA,
        ],
    )
    def scatter(e_hbm, dst_hbm, out_hbm, acc_sh, ebuf0, ebuf1,
                idx0, idx1,
                esem0, esem1, isem0, isem1, ssem0, ssem1):
        cid = lax.axis_index("c")
        sid = lax.axis_index("s")
        wid = sid * _NC + cid
        wb_e = wid * epw
        wb_d = eoff + wid * epw
        ebufs = (ebuf0, ebuf1)
        idxs = (idx0, idx1)
        esems = (esem0, esem1)
        isems = (isem0, isem1)
        ssems = (ssem0, ssem1)

        def zb(i, carry):
            r = i // (D // 16)
            c2 = (i % (D // 16)) * 16
            ebuf0[r, pl.ds(c2, 16)] = jnp.zeros((16,), jnp.float32)
            return carry

        lax.fori_loop(0, se * (D // 16), zb, 0)
        zrep = -(-ZC // se)

        def zc(z, carry):
            ci = z * _NS + sid

            @pl.when(ci < NZ)
            def _():
                for j in range(zrep):
                    base = pl.multiple_of(ci * ZC + j * se, 8)
                    pltpu.sync_copy(ebuf0,
                                    acc_sh.at[pl.ds(base, se)])

                return None

            return carry

        lax.fori_loop(0, ZPW, zc, 0)
        plsc.subcore_barrier()

        def load(k, b):
            pltpu.async_copy(e_hbm.at[pl.ds(wb_e + k * se, se)],
                             ebufs[b], esems[b])
            pltpu.async_copy(dst_hbm.at[pl.ds(wb_d + k * se, se)],
                             idxs[b], isems[b])

        load(0, 0)

        def step(k, b):
            o = 1 - b
            pltpu.make_async_copy(e_hbm.at[pl.ds(0, se)], ebufs[b],
                                  esems[b]).wait()
            pltpu.make_async_copy(dst_hbm.at[pl.ds(0, se)], idxs[b],
                                  isems[b]).wait()
            pltpu.async_copy(ebufs[b], acc_sh.at[idxs[b]], ssems[b],
                             add=True)

            @pl.when(k >= 1)
            def _():
                pltpu.make_async_copy(ebufs[o], acc_sh.at[idxs[o]],
                                      ssems[o]).wait()

            @pl.when(k + 1 < sn)
            def _():
                load(k + 1, o)

        def body(k2, carry):
            step(k2 * 2, 0)
            step(k2 * 2 + 1, 1)
            return carry

        lax.fori_loop(0, sn // 2, body, 0)

        if sn % 2 == 1:
            pltpu.make_async_copy(e_hbm.at[pl.ds(0, se)], ebuf0,
                                  esem0).wait()
            pltpu.make_async_copy(dst_hbm.at[pl.ds(0, se)], idx0,
                                  isem0).wait()
            pltpu.async_copy(ebuf0, acc_sh.at[idx0], ssem0, add=True)
            pltpu.make_async_copy(ebuf1, acc_sh.at[idx1], ssem1).wait()
            pltpu.make_async_copy(ebuf0, acc_sh.at[idx0], ssem0).wait()
        else:
            pltpu.make_async_copy(ebuf1, acc_sh.at[idx1], ssem1).wait()
        plsc.subcore_barrier()

        def oc(z, carry):
            ci = z * _NS + sid

            @pl.when(ci < NZ)
            def _():
                base = pl.multiple_of(ci * ZC, 8)
                pltpu.sync_copy(acc_sh.at[pl.ds(base, ZC)],
                                out_hbm.at[cid, pl.ds(base, ZC)])

            return carry

        lax.fori_loop(0, ZPW, oc, 0)

    return scatter


_gather_h1 = _make_gather(0, E_H1)
_gather_h2 = _make_gather(E_H1, E_H2)
_scatter_h1 = _make_scatter(0, E_H1, 80)
_scatter_h2 = _make_scatter(E_H1, E_H2, 40)


def _ln(y, g, bt):
    mu = jnp.mean(y, axis=-1, keepdims=True)
    var = jnp.mean((y - mu) * (y - mu), axis=-1, keepdims=True)
    return (y - mu) * lax.rsqrt(var + 1e-5) * g + bt


def _embed_body(x_ref, w0, b0, w1, b1, g, bt, o_ref):
    h = jax.nn.silu(jnp.dot(x_ref[...], w0[...],
                            preferred_element_type=jnp.float32) + b0[...])
    y = jnp.dot(h, w1[...], preferred_element_type=jnp.float32) + b1[...]
    o_ref[...] = _ln(y, g[...], bt[...])


def _pq_body(n_ref, ws, wd, p_ref, q_ref):
    x = n_ref[...]
    p_ref[...] = jnp.dot(x, ws[...], preferred_element_type=jnp.float32)
    q_ref[...] = jnp.dot(x, wd[...], preferred_element_type=jnp.float32)


def _edge_body(gs_ref, gd_ref, e_ref, we, b0, w1, b1, g, bt, o_ref):
    x = e_ref[...]
    pre = gs_ref[...] + gd_ref[...] + jnp.dot(
        x, we[...], preferred_element_type=jnp.float32) + b0[...]
    h = jax.nn.silu(pre)
    y = jnp.dot(h, w1[...], preferred_element_type=jnp.float32) + b1[...]
    o_ref[...] = x + _ln(y, g[...], bt[...])


def _node_body(a_ref, c_ref, n_ref, wa, wn, b0, w1, b1, g, bt, o_ref):
    agg = a_ref[0] + a_ref[1] + c_ref[0] + c_ref[1]
    x = n_ref[...]
    pre = (jnp.dot(agg, wa[...], preferred_element_type=jnp.float32)
           + jnp.dot(x, wn[...], preferred_element_type=jnp.float32)
           + b0[...])
    h = jax.nn.silu(pre)
    y = jnp.dot(h, w1[...], preferred_element_type=jnp.float32) + b1[...]
    o_ref[...] = x + _ln(y, g[...], bt[...])


_E_TILE = 640
_N_TILE = 1000


def _full(shape):
    return pl.BlockSpec(shape, lambda i: (0,) * len(shape))


def _tc_embed(x, ecnt, w0, b0, w1, b1, g, bt):
    grid = (ecnt // _E_TILE,)
    return pl.pallas_call(
        _embed_body,
        grid=grid,
        in_specs=[
            pl.BlockSpec((_E_TILE, 4), lambda i: (i, 0)),
            _full((4, D)), _full((1, D)), _full((D, D)), _full((1, D)),
            _full((1, D)), _full((1, D)),
        ],
        out_specs=pl.BlockSpec((_E_TILE, D), lambda i: (i, 0)),
        out_shape=jax.ShapeDtypeStruct((ecnt, D), jnp.float32),
    )(x, w0, b0, w1, b1, g, bt)


def _tc_pq(nfeat, ws, wd):
    grid = (N_NODES // _N_TILE,)
    return pl.pallas_call(
        _pq_body,
        grid=grid,
        in_specs=[
            pl.BlockSpec((_N_TILE, D), lambda i: (i, 0)),
            _full((D, D)), _full((D, D)),
        ],
        out_specs=[pl.BlockSpec((_N_TILE, D), lambda i: (i, 0)),
                   pl.BlockSpec((_N_TILE, D), lambda i: (i, 0))],
        out_shape=[jax.ShapeDtypeStruct((N_NODES, D), jnp.float32),
                   jax.ShapeDtypeStruct((N_NODES, D), jnp.float32)],
    )(nfeat, ws, wd)


def _tc_edge(gs, gd, efeat, we, b0, w1, b1, g, bt):
    ecnt = gs.shape[0]
    grid = (ecnt // _E_TILE,)
    return pl.pallas_call(
        _edge_body,
        grid=grid,
        in_specs=[
            pl.BlockSpec((_E_TILE, D), lambda i: (i, 0)),
            pl.BlockSpec((_E_TILE, D), lambda i: (i, 0)),
            pl.BlockSpec((_E_TILE, D), lambda i: (i, 0)),
            _full((D, D)), _full((1, D)), _full((D, D)), _full((1, D)),
            _full((1, D)), _full((1, D)),
        ],
        out_specs=pl.BlockSpec((_E_TILE, D), lambda i: (i, 0)),
        out_shape=jax.ShapeDtypeStruct((ecnt, D), jnp.float32),
    )(gs, gd, efeat, we, b0, w1, b1, g, bt)


def _tc_node(a2, c2, nfeat, wa, wn, b0, w1, b1, g, bt):
    grid = (N_NODES // _N_TILE,)
    return pl.pallas_call(
        _node_body,
        grid=grid,
        in_specs=[
            pl.BlockSpec((_NC, _N_TILE, D), lambda i: (0, i, 0)),
            pl.BlockSpec((_NC, _N_TILE, D), lambda i: (0, i, 0)),
            pl.BlockSpec((_N_TILE, D), lambda i: (i, 0)),
            _full((D, D)), _full((D, D)), _full((1, D)), _full((D, D)),
            _full((1, D)), _full((1, D)), _full((1, D)),
        ],
        out_specs=pl.BlockSpec((_N_TILE, D), lambda i: (i, 0)),
        out_shape=jax.ShapeDtypeStruct((N_NODES, D), jnp.float32),
    )(a2, c2, nfeat, wa, wn, b0, w1, b1, g, bt)


def kernel(mesh_nfeat, edge_index, mesh_efeat,
           emb_W0, emb_b0, emb_W1, emb_b1, emb_g, emb_bt,
           We0, be0, We1, be1, eg, ebt,
           Wn0, bn0, Wn1, bn1, ng, nbt):
    r = lambda v: v.reshape(1, D)
    src = edge_index[0].astype(jnp.int32)
    dst = edge_index[1].astype(jnp.int32)

    ef1 = _tc_embed(mesh_efeat[:E_H1], E_H1, emb_W0, r(emb_b0), emb_W1,
                    r(emb_b1), r(emb_g), r(emb_bt))
    ef2 = _tc_embed(mesh_efeat[E_H1:], E_H2, emb_W0, r(emb_b0), emb_W1,
                    r(emb_b1), r(emb_g), r(emb_bt))
    nfeat = mesh_nfeat
    for i in range(N_BLOCK):
        p, q = _tc_pq(nfeat, We0[i, :D], We0[i, D:2 * D])
        gs1, gd1 = _gather_h1(p, q, src, dst)
        gs2, gd2 = _gather_h2(p, q, src, dst)
        ef1 = _tc_edge(gs1, gd1, ef1, We0[i, 2 * D:], r(be0[i]),
                       We1[i], r(be1[i]), r(eg[i]), r(ebt[i]))
        a2 = _scatter_h1(ef1, dst)
        ef2 = _tc_edge(gs2, gd2, ef2, We0[i, 2 * D:], r(be0[i]),
                       We1[i], r(be1[i]), r(eg[i]), r(ebt[i]))
        c2 = _scatter_h2(ef2, dst)
        nfeat = _tc_node(a2, c2, nfeat, Wn0[i, :D], Wn0[i, D:],
                         r(bn0[i]), Wn1[i], r(bn1[i]), r(ng[i]),
                         r(nbt[i]))
    return (nfeat, jnp.concatenate([ef1, ef2], axis=0))


# speedup vs baseline: 1.2821x; 1.2821x over previous
import functools

import jax
import jax.numpy as jnp
from jax import lax
from jax.experimental import pallas as pl
from jax.experimental.pallas import tpu as pltpu
from jax.experimental.pallas import tpu_sc as plsc

N_NODES = 10000
N_EDGES = 160000
D = 128
N_BLOCK = 4

_NC = 2
_NS = 16
_NW = _NC * _NS

E_H1 = 81920
E_H2 = N_EDGES - E_H1

ZC = 80
NZ = N_NODES // ZC
ZPW = -(-NZ // _NS)

_mesh = plsc.VectorSubcoreMesh(core_axis_name="c", subcore_axis_name="s")

GE = 80


def _make_gather(eoff, ecnt):
    ept = ecnt // _NS
    gn = ept // GE
    assert ept * _NS == ecnt and gn * GE == ept and ept % 8 == 0

    @functools.partial(
        pl.kernel,
        out_type=(jax.ShapeDtypeStruct((ecnt, D), jnp.float32),
                  jax.ShapeDtypeStruct((ecnt, D), jnp.float32)),
        mesh=_mesh,
        scratch_types=[
            pltpu.VMEM_SHARED((N_NODES, D), jnp.float32),
            pltpu.VMEM((GE,), jnp.int32),
            pltpu.VMEM((GE,), jnp.int32),
            pltpu.VMEM((GE, D), jnp.float32),
            pltpu.VMEM((GE, D), jnp.float32),
            pltpu.SemaphoreType.DMA,
            pltpu.SemaphoreType.DMA,
            pltpu.SemaphoreType.DMA,
            pltpu.SemaphoreType.DMA,
            pltpu.SemaphoreType.DMA,
            pltpu.SemaphoreType.DMA,
        ],
    )
    def gather(p_hbm, q_hbm, src_hbm, dst_hbm, gs_hbm, gd_hbm,
               tbl_sh, idx0, idx1, buf0, buf1,
               isem0, isem1, gsem0, gsem1, wsem0, wsem1):
        cid = lax.axis_index("c")
        sid = lax.axis_index("s")
        idxs = (idx0, idx1)
        bufs = (buf0, buf1)
        isems = (isem0, isem1)
        gsems = (gsem0, gsem1)
        wsems = (wsem0, wsem1)

        def stage(tab_hbm):
            def st(z, carry):
                ci = z * _NS + sid

                @pl.when(ci < NZ)
                def _():
                    base = pl.multiple_of(ci * ZC, 8)
                    pltpu.sync_copy(tab_hbm.at[pl.ds(base, ZC)],
                                    tbl_sh.at[pl.ds(base, ZC)])

                return carry

            lax.fori_loop(0, ZPW, st, 0)

        @pl.when(cid == 0)
        def _():
            stage(p_hbm)

        @pl.when(cid == 1)
        def _():
            stage(q_hbm)

        plsc.subcore_barrier()

        def run(ind_hbm, out_hbm):
            tb_i = eoff + sid * ept
            tb_o = sid * ept

            def idxload(k, b):
                pltpu.async_copy(ind_hbm.at[pl.ds(tb_i + k * GE, GE)],
                                 idxs[b], isems[b])

            idxload(0, 0)
            idxload(1, 1)
            pltpu.make_async_copy(ind_hbm.at[pl.ds(0, GE)], idx0,
                                  isem0).wait()
            pltpu.async_copy(tbl_sh.at[idx0], buf0, gsem0)

            def step(k, b):
                o = 1 - b
                pltpu.make_async_copy(tbl_sh.at[idxs[b]], bufs[b],
                                      gsems[b]).wait()

                @pl.when(k + 2 < gn)
                def _():
                    idxload(k + 2, b)

                pltpu.async_copy(bufs[b],
                                 out_hbm.at[pl.ds(tb_o + k * GE, GE)],
                                 wsems[b])
                pltpu.make_async_copy(ind_hbm.at[pl.ds(0, GE)], idxs[o],
                                      isems[o]).wait()

                @pl.when(k >= 1)
                def _():
                    pltpu.make_async_copy(bufs[o],
                                          out_hbm.at[pl.ds(0, GE)],
                                          wsems[o]).wait()

                pltpu.async_copy(tbl_sh.at[idxs[o]], bufs[o], gsems[o])

            def body(k2, carry):
                step(k2 * 2, 0)
                step(k2 * 2 + 1, 1)
                return carry

            lax.fori_loop(0, (gn - 1) // 2, body, 0)

            if gn % 2 == 1:
                pltpu.make_async_copy(tbl_sh.at[idx0], buf0,
                                      gsem0).wait()
                pltpu.async_copy(
                    buf0, out_hbm.at[pl.ds(tb_o + (gn - 1) * GE, GE)],
                    wsem0)
                pltpu.make_async_copy(buf1, out_hbm.at[pl.ds(0, GE)],
                                      wsem1).wait()
                pltpu.make_async_copy(buf0, out_hbm.at[pl.ds(0, GE)],
                                      wsem0).wait()
            else:
                pltpu.make_async_copy(tbl_sh.at[idx0], buf0,
                                      gsem0).wait()
                pltpu.async_copy(
                    buf0, out_hbm.at[pl.ds(tb_o + (gn - 2) * GE, GE)],
                    wsem0)
                pltpu.make_async_copy(ind_hbm.at[pl.ds(0, GE)], idx1,
                                      isem1).wait()
                pltpu.make_async_copy(buf1, out_hbm.at[pl.ds(0, GE)],
                                      wsem1).wait()
                pltpu.async_copy(tbl_sh.at[idx1], buf1, gsem1)
                pltpu.make_async_copy(tbl_sh.at[idx1], buf1,
                                      gsem1).wait()
                pltpu.async_copy(
                    buf1, out_hbm.at[pl.ds(tb_o + (gn - 1) * GE, GE)],
                    wsem1)
                pltpu.make_async_copy(buf0, out_hbm.at[pl.ds(0, GE)],
                                      wsem0).wait()
                pltpu.make_async_copy(buf1, out_hbm.at[pl.ds(0, GE)],
                                      wsem1).wait()

        @pl.when(cid == 0)
        def _():
            run(src_hbm, gs_hbm)

        @pl.when(cid == 1)
        def _():
            run(dst_hbm, gd_hbm)

    return gather


def _make_scatter(eoff, ecnt, se):
    epw = ecnt // _NW
    sn = epw // se
    assert epw * _NW == ecnt and sn * se == epw and epw % 8 == 0
    assert sn >= 2 and se % 8 == 0

    @functools.partial(
        pl.kernel,
        out_type=jax.ShapeDtypeStruct((_NC, N_NODES, D), jnp.float32),
        mesh=_mesh,
        scratch_types=[
            pltpu.VMEM_SHARED((N_NODES, D), jnp.float32),
            pltpu.VMEM((se, D), jnp.float32),
            pltpu.VMEM((se, D), jnp.float32),
            pltpu.VMEM((se,), jnp.int32),
            pltpu.VMEM((se,), jnp.int32),
            pltpu.SemaphoreType.DMA,
            pltpu.SemaphoreType.DMA,
            pltpu.SemaphoreType.DMA,
            pltpu.SemaphoreType.DMA,
            pltpu.SemaphoreType.DMA,
            pltpu.SemaphoreType.DMA,
        ],
    )
    def scatter(e_hbm, dst_hbm, out_hbm, acc_sh, ebuf0, ebuf1,
                idx0, idx1,
                esem0, esem1, isem0, isem1, ssem0, ssem1):
        cid = lax.axis_index("c")
        sid = lax.axis_index("s")
        wid = sid * _NC + cid
        wb_e = wid * epw
        wb_d = eoff + wid * epw
        ebufs = (ebuf0, ebuf1)
        idxs = (idx0, idx1)
        esems = (esem0, esem1)
        isems = (isem0, isem1)
        ssems = (ssem0, ssem1)

        def zb(i, carry):
            r = i // (D // 16)
            c2 = (i % (D // 16)) * 16
            ebuf0[r, pl.ds(c2, 16)] = jnp.zeros((16,), jnp.float32)
            return carry

        lax.fori_loop(0, se * (D // 16), zb, 0)
        zrep = -(-ZC // se)

        def zc(z, carry):
            ci = z * _NS + sid

            @pl.when(ci < NZ)
            def _():
                for j in range(zrep):
                    base = pl.multiple_of(ci * ZC + j * se, 8)
                    pltpu.sync_copy(ebuf0,
                                    acc_sh.at[pl.ds(base, se)])

                return None

            return carry

        lax.fori_loop(0, ZPW, zc, 0)
        plsc.subcore_barrier()

        def load(k, b):
            pltpu.async_copy(e_hbm.at[pl.ds(wb_e + k * se, se)],
                             ebufs[b], esems[b])
            pltpu.async_copy(dst_hbm.at[pl.ds(wb_d + k * se, se)],
                             idxs[b], isems[b])

        load(0, 0)

        def step(k, b):
            o = 1 - b
            pltpu.make_async_copy(e_hbm.at[pl.ds(0, se)], ebufs[b],
                                  esems[b]).wait()
            pltpu.make_async_copy(dst_hbm.at[pl.ds(0, se)], idxs[b],
                                  isems[b]).wait()
            pltpu.async_copy(ebufs[b], acc_sh.at[idxs[b]], ssems[b],
                             add=True)

            @pl.when(k >= 1)
            def _():
                pltpu.make_async_copy(ebufs[o], acc_sh.at[idxs[o]],
                                      ssems[o]).wait()

            @pl.when(k + 1 < sn)
            def _():
                load(k + 1, o)

        def body(k2, carry):
            step(k2 * 2, 0)
            step(k2 * 2 + 1, 1)
            return carry

        lax.fori_loop(0, sn // 2, body, 0)

        if sn % 2 == 1:
            pltpu.make_async_copy(e_hbm.at[pl.ds(0, se)], ebuf0,
                                  esem0).wait()
            pltpu.make_async_copy(dst_hbm.at[pl.ds(0, se)], idx0,
                                  isem0).wait()
            pltpu.async_copy(ebuf0, acc_sh.at[idx0], ssem0, add=True)
            pltpu.make_async_copy(ebuf1, acc_sh.at[idx1], ssem1).wait()
            pltpu.make_async_copy(ebuf0, acc_sh.at[idx0], ssem0).wait()
        else:
            pltpu.make_async_copy(ebuf1, acc_sh.at[idx1], ssem1).wait()
        plsc.subcore_barrier()

        def oc(z, carry):
            ci = z * _NS + sid

            @pl.when(ci < NZ)
            def _():
                base = pl.multiple_of(ci * ZC, 8)
                pltpu.sync_copy(acc_sh.at[pl.ds(base, ZC)],
                                out_hbm.at[cid, pl.ds(base, ZC)])

            return carry

        lax.fori_loop(0, ZPW, oc, 0)

    return scatter


_gather_h1 = _make_gather(0, E_H1)
_gather_h2 = _make_gather(E_H1, E_H2)
_scatter_h1 = _make_scatter(0, E_H1, 80)
_scatter_h2 = _make_scatter(E_H1, E_H2, 40)


def _ln(y, g, bt):
    mu = jnp.mean(y, axis=-1, keepdims=True)
    var = jnp.mean((y - mu) * (y - mu), axis=-1, keepdims=True)
    return (y - mu) * lax.rsqrt(var + 1e-5) * g + bt


def _embed_body(x_ref, w0, b0, w1, b1, g, bt, o_ref):
    h = jax.nn.silu(jnp.dot(x_ref[...], w0[...],
                            preferred_element_type=jnp.float32) + b0[...])
    y = jnp.dot(h, w1[...], preferred_element_type=jnp.float32) + b1[...]
    o_ref[...] = _ln(y, g[...], bt[...])


def _pq_body(n_ref, ws, wd, p_ref, q_ref):
    x = n_ref[...]
    p_ref[...] = jnp.dot(x, ws[...], preferred_element_type=jnp.float32)
    q_ref[...] = jnp.dot(x, wd[...], preferred_element_type=jnp.float32)


def _edge_body(gs_ref, gd_ref, e_ref, we, b0, w1, b1, g, bt, o_ref):
    x = e_ref[...]
    pre = gs_ref[...] + gd_ref[...] + jnp.dot(
        x, we[...], preferred_element_type=jnp.float32) + b0[...]
    h = jax.nn.silu(pre)
    y = jnp.dot(h, w1[...], preferred_element_type=jnp.float32) + b1[...]
    o_ref[...] = x + _ln(y, g[...], bt[...])


def _node_body(a_ref, c_ref, n_ref, wa, wn, b0, w1, b1, g, bt, o_ref):
    agg = a_ref[0] + a_ref[1] + c_ref[0] + c_ref[1]
    x = n_ref[...]
    pre = (jnp.dot(agg, wa[...], preferred_element_type=jnp.float32)
           + jnp.dot(x, wn[...], preferred_element_type=jnp.float32)
           + b0[...])
    h = jax.nn.silu(pre)
    y = jnp.dot(h, w1[...], preferred_element_type=jnp.float32) + b1[...]
    o_ref[...] = x + _ln(y, g[...], bt[...])


_E_TILE = 1280
_N_TILE = 2000


def _full(shape):
    return pl.BlockSpec(shape, lambda i: (0,) * len(shape))


def _tc_embed(x, ecnt, w0, b0, w1, b1, g, bt):
    grid = (ecnt // _E_TILE,)
    return pl.pallas_call(
        _embed_body,
        grid=grid,
        in_specs=[
            pl.BlockSpec((_E_TILE, 4), lambda i: (i, 0)),
            _full((4, D)), _full((1, D)), _full((D, D)), _full((1, D)),
            _full((1, D)), _full((1, D)),
        ],
        out_specs=pl.BlockSpec((_E_TILE, D), lambda i: (i, 0)),
        out_shape=jax.ShapeDtypeStruct((ecnt, D), jnp.float32),
    )(x, w0, b0, w1, b1, g, bt)


def _tc_pq(nfeat, ws, wd):
    grid = (N_NODES // _N_TILE,)
    return pl.pallas_call(
        _pq_body,
        grid=grid,
        in_specs=[
            pl.BlockSpec((_N_TILE, D), lambda i: (i, 0)),
            _full((D, D)), _full((D, D)),
        ],
        out_specs=[pl.BlockSpec((_N_TILE, D), lambda i: (i, 0)),
                   pl.BlockSpec((_N_TILE, D), lambda i: (i, 0))],
        out_shape=[jax.ShapeDtypeStruct((N_NODES, D), jnp.float32),
                   jax.ShapeDtypeStruct((N_NODES, D), jnp.float32)],
    )(nfeat, ws, wd)


def _tc_edge(gs, gd, efeat, we, b0, w1, b1, g, bt):
    ecnt = gs.shape[0]
    grid = (ecnt // _E_TILE,)
    return pl.pallas_call(
        _edge_body,
        grid=grid,
        in_specs=[
            pl.BlockSpec((_E_TILE, D), lambda i: (i, 0)),
            pl.BlockSpec((_E_TILE, D), lambda i: (i, 0)),
            pl.BlockSpec((_E_TILE, D), lambda i: (i, 0)),
            _full((D, D)), _full((1, D)), _full((D, D)), _full((1, D)),
            _full((1, D)), _full((1, D)),
        ],
        out_specs=pl.BlockSpec((_E_TILE, D), lambda i: (i, 0)),
        out_shape=jax.ShapeDtypeStruct((ecnt, D), jnp.float32),
    )(gs, gd, efeat, we, b0, w1, b1, g, bt)


def _tc_node(a2, c2, nfeat, wa, wn, b0, w1, b1, g, bt):
    grid = (N_NODES // _N_TILE,)
    return pl.pallas_call(
        _node_body,
        grid=grid,
        in_specs=[
            pl.BlockSpec((_NC, _N_TILE, D), lambda i: (0, i, 0)),
            pl.BlockSpec((_NC, _N_TILE, D), lambda i: (0, i, 0)),
            pl.BlockSpec((_N_TILE, D), lambda i: (i, 0)),
            _full((D, D)), _full((D, D)), _full((1, D)), _full((D, D)),
            _full((1, D)), _full((1, D)), _full((1, D)),
        ],
        out_specs=pl.BlockSpec((_N_TILE, D), lambda i: (i, 0)),
        out_shape=jax.ShapeDtypeStruct((N_NODES, D), jnp.float32),
    )(a2, c2, nfeat, wa, wn, b0, w1, b1, g, bt)


def kernel(mesh_nfeat, edge_index, mesh_efeat,
           emb_W0, emb_b0, emb_W1, emb_b1, emb_g, emb_bt,
           We0, be0, We1, be1, eg, ebt,
           Wn0, bn0, Wn1, bn1, ng, nbt):
    r = lambda v: v.reshape(1, D)
    src = edge_index[0].astype(jnp.int32)
    dst = edge_index[1].astype(jnp.int32)

    ef1 = _tc_embed(mesh_efeat[:E_H1], E_H1, emb_W0, r(emb_b0), emb_W1,
                    r(emb_b1), r(emb_g), r(emb_bt))
    ef2 = _tc_embed(mesh_efeat[E_H1:], E_H2, emb_W0, r(emb_b0), emb_W1,
                    r(emb_b1), r(emb_g), r(emb_bt))
    nfeat = mesh_nfeat
    for i in range(N_BLOCK):
        p, q = _tc_pq(nfeat, We0[i, :D], We0[i, D:2 * D])
        gs1, gd1 = _gather_h1(p, q, src, dst)
        gs2, gd2 = _gather_h2(p, q, src, dst)
        ef1 = _tc_edge(gs1, gd1, ef1, We0[i, 2 * D:], r(be0[i]),
                       We1[i], r(be1[i]), r(eg[i]), r(ebt[i]))
        a2 = _scatter_h1(ef1, dst)
        ef2 = _tc_edge(gs2, gd2, ef2, We0[i, 2 * D:], r(be0[i]),
                       We1[i], r(be1[i]), r(eg[i]), r(ebt[i]))
        c2 = _scatter_h2(ef2, dst)
        nfeat = _tc_node(a2, c2, nfeat, Wn0[i, :D], Wn0[i, D:],
                         r(bn0[i]), Wn1[i], r(bn1[i]), r(ng[i]),
                         r(nbt[i]))
    return (nfeat, jnp.concatenate([ef1, ef2], axis=0))


# speedup vs baseline: 1.3865x; 1.0814x over previous
import functools

import jax
import jax.numpy as jnp
from jax import lax
from jax.experimental import pallas as pl
from jax.experimental.pallas import tpu as pltpu
from jax.experimental.pallas import tpu_sc as plsc

N_NODES = 10000
N_EDGES = 160000
D = 128
N_BLOCK = 4

_NC = 2
_NS = 16
_NW = _NC * _NS

E_H1 = 81920
E_H2 = N_EDGES - E_H1

ZC = 80
NZ = N_NODES // ZC
ZPW = -(-NZ // _NS)

_mesh = plsc.VectorSubcoreMesh(core_axis_name="c", subcore_axis_name="s")

GE = 80


def _make_gather(eoff, ecnt):
    ept = ecnt // _NS
    gn = ept // GE
    assert ept * _NS == ecnt and gn * GE == ept and ept % 8 == 0

    @functools.partial(
        pl.kernel,
        out_type=(jax.ShapeDtypeStruct((ecnt, D), jnp.float32),
                  jax.ShapeDtypeStruct((ecnt, D), jnp.float32)),
        mesh=_mesh,
        scratch_types=[
            pltpu.VMEM_SHARED((N_NODES, D), jnp.float32),
            pltpu.VMEM((GE,), jnp.int32),
            pltpu.VMEM((GE,), jnp.int32),
            pltpu.VMEM((GE, D), jnp.float32),
            pltpu.VMEM((GE, D), jnp.float32),
            pltpu.SemaphoreType.DMA,
            pltpu.SemaphoreType.DMA,
            pltpu.SemaphoreType.DMA,
            pltpu.SemaphoreType.DMA,
            pltpu.SemaphoreType.DMA,
            pltpu.SemaphoreType.DMA,
        ],
    )
    def gather(p_hbm, q_hbm, src_hbm, dst_hbm, gs_hbm, gd_hbm,
               tbl_sh, idx0, idx1, buf0, buf1,
               isem0, isem1, gsem0, gsem1, wsem0, wsem1):
        cid = lax.axis_index("c")
        sid = lax.axis_index("s")
        idxs = (idx0, idx1)
        bufs = (buf0, buf1)
        isems = (isem0, isem1)
        gsems = (gsem0, gsem1)
        wsems = (wsem0, wsem1)

        def stage(tab_hbm):
            def st(z, carry):
                ci = z * _NS + sid

                @pl.when(ci < NZ)
                def _():
                    base = pl.multiple_of(ci * ZC, 8)
                    pltpu.sync_copy(tab_hbm.at[pl.ds(base, ZC)],
                                    tbl_sh.at[pl.ds(base, ZC)])

                return carry

            lax.fori_loop(0, ZPW, st, 0)

        @pl.when(cid == 0)
        def _():
            stage(p_hbm)

        @pl.when(cid == 1)
        def _():
            stage(q_hbm)

        plsc.subcore_barrier()

        def run(ind_hbm, out_hbm):
            tb_i = eoff + sid * ept
            tb_o = sid * ept

            def idxload(k, b):
                pltpu.async_copy(ind_hbm.at[pl.ds(tb_i + k * GE, GE)],
                                 idxs[b], isems[b])

            idxload(0, 0)
            idxload(1, 1)
            pltpu.make_async_copy(ind_hbm.at[pl.ds(0, GE)], idx0,
                                  isem0).wait()
            pltpu.async_copy(tbl_sh.at[idx0], buf0, gsem0)

            def step(k, b):
                o = 1 - b
                pltpu.make_async_copy(tbl_sh.at[idxs[b]], bufs[b],
                                      gsems[b]).wait()

                @pl.when(k + 2 < gn)
                def _():
                    idxload(k + 2, b)

                pltpu.async_copy(bufs[b],
                                 out_hbm.at[pl.ds(tb_o + k * GE, GE)],
                                 wsems[b])
                pltpu.make_async_copy(ind_hbm.at[pl.ds(0, GE)], idxs[o],
                                      isems[o]).wait()

                @pl.when(k >= 1)
                def _():
                    pltpu.make_async_copy(bufs[o],
                                          out_hbm.at[pl.ds(0, GE)],
                                          wsems[o]).wait()

                pltpu.async_copy(tbl_sh.at[idxs[o]], bufs[o], gsems[o])

            def body(k2, carry):
                step(k2 * 2, 0)
                step(k2 * 2 + 1, 1)
                return carry

            lax.fori_loop(0, (gn - 1) // 2, body, 0)

            if gn % 2 == 1:
                pltpu.make_async_copy(tbl_sh.at[idx0], buf0,
                                      gsem0).wait()
                pltpu.async_copy(
                    buf0, out_hbm.at[pl.ds(tb_o + (gn - 1) * GE, GE)],
                    wsem0)
                pltpu.make_async_copy(buf1, out_hbm.at[pl.ds(0, GE)],
                                      wsem1).wait()
                pltpu.make_async_copy(buf0, out_hbm.at[pl.ds(0, GE)],
                                      wsem0).wait()
            else:
                pltpu.make_async_copy(tbl_sh.at[idx0], buf0,
                                      gsem0).wait()
                pltpu.async_copy(
                    buf0, out_hbm.at[pl.ds(tb_o + (gn - 2) * GE, GE)],
                    wsem0)
                pltpu.make_async_copy(ind_hbm.at[pl.ds(0, GE)], idx1,
                                      isem1).wait()
                pltpu.make_async_copy(buf1, out_hbm.at[pl.ds(0, GE)],
                                      wsem1).wait()
                pltpu.async_copy(tbl_sh.at[idx1], buf1, gsem1)
                pltpu.make_async_copy(tbl_sh.at[idx1], buf1,
                                      gsem1).wait()
                pltpu.async_copy(
                    buf1, out_hbm.at[pl.ds(tb_o + (gn - 1) * GE, GE)],
                    wsem1)
                pltpu.make_async_copy(buf0, out_hbm.at[pl.ds(0, GE)],
                                      wsem0).wait()
                pltpu.make_async_copy(buf1, out_hbm.at[pl.ds(0, GE)],
                                      wsem1).wait()

        @pl.when(cid == 0)
        def _():
            run(src_hbm, gs_hbm)

        @pl.when(cid == 1)
        def _():
            run(dst_hbm, gd_hbm)

    return gather


def _make_scatter(eoff, ecnt, se):
    epw = ecnt // _NW
    sn = epw // se
    assert epw * _NW == ecnt and sn * se == epw and epw % 8 == 0
    assert sn >= 2 and se % 8 == 0

    @functools.partial(
        pl.kernel,
        out_type=jax.ShapeDtypeStruct((_NC, N_NODES, D), jnp.float32),
        mesh=_mesh,
        scratch_types=[
            pltpu.VMEM_SHARED((N_NODES, D), jnp.float32),
            pltpu.VMEM((se, D), jnp.float32),
            pltpu.VMEM((se, D), jnp.float32),
            pltpu.VMEM((se,), jnp.int32),
            pltpu.VMEM((se,), jnp.int32),
            pltpu.SemaphoreType.DMA,
            pltpu.SemaphoreType.DMA,
            pltpu.SemaphoreType.DMA,
            pltpu.SemaphoreType.DMA,
            pltpu.SemaphoreType.DMA,
            pltpu.SemaphoreType.DMA,
        ],
    )
    def scatter(e_hbm, dst_hbm, out_hbm, acc_sh, ebuf0, ebuf1,
                idx0, idx1,
                esem0, esem1, isem0, isem1, ssem0, ssem1):
        cid = lax.axis_index("c")
        sid = lax.axis_index("s")
        wid = sid * _NC + cid
        wb_e = wid * epw
        wb_d = eoff + wid * epw
        ebufs = (ebuf0, ebuf1)
        idxs = (idx0, idx1)
        esems = (esem0, esem1)
        isems = (isem0, isem1)
        ssems = (ssem0, ssem1)

        def zb(i, carry):
            r = i // (D // 16)
            c2 = (i % (D // 16)) * 16
            ebuf0[r, pl.ds(c2, 16)] = jnp.zeros((16,), jnp.float32)
            return carry

        lax.fori_loop(0, se * (D // 16), zb, 0)
        zrep = -(-ZC // se)

        def zc(z, carry):
            ci = z * _NS + sid

            @pl.when(ci < NZ)
            def _():
                for j in range(zrep):
                    base = pl.multiple_of(ci * ZC + j * se, 8)
                    pltpu.sync_copy(ebuf0,
                                    acc_sh.at[pl.ds(base, se)])

                return None

            return carry

        lax.fori_loop(0, ZPW, zc, 0)
        plsc.subcore_barrier()

        def load(k, b):
            pltpu.async_copy(e_hbm.at[pl.ds(wb_e + k * se, se)],
                             ebufs[b], esems[b])
            pltpu.async_copy(dst_hbm.at[pl.ds(wb_d + k * se, se)],
                             idxs[b], isems[b])

        load(0, 0)

        def step(k, b):
            o = 1 - b
            pltpu.make_async_copy(e_hbm.at[pl.ds(0, se)], ebufs[b],
                                  esems[b]).wait()
            pltpu.make_async_copy(dst_hbm.at[pl.ds(0, se)], idxs[b],
                                  isems[b]).wait()
            pltpu.async_copy(ebufs[b], acc_sh.at[idxs[b]], ssems[b],
                             add=True)

            @pl.when(k >= 1)
            def _():
                pltpu.make_async_copy(ebufs[o], acc_sh.at[idxs[o]],
                                      ssems[o]).wait()

            @pl.when(k + 1 < sn)
            def _():
                load(k + 1, o)

        def body(k2, carry):
            step(k2 * 2, 0)
            step(k2 * 2 + 1, 1)
            return carry

        lax.fori_loop(0, sn // 2, body, 0)

        if sn % 2 == 1:
            pltpu.make_async_copy(e_hbm.at[pl.ds(0, se)], ebuf0,
                                  esem0).wait()
            pltpu.make_async_copy(dst_hbm.at[pl.ds(0, se)], idx0,
                                  isem0).wait()
            pltpu.async_copy(ebuf0, acc_sh.at[idx0], ssem0, add=True)
            pltpu.make_async_copy(ebuf1, acc_sh.at[idx1], ssem1).wait()
            pltpu.make_async_copy(ebuf0, acc_sh.at[idx0], ssem0).wait()
        else:
            pltpu.make_async_copy(ebuf1, acc_sh.at[idx1], ssem1).wait()
        plsc.subcore_barrier()

        def oc(z, carry):
            ci = z * _NS + sid

            @pl.when(ci < NZ)
            def _():
                base = pl.multiple_of(ci * ZC, 8)
                pltpu.sync_copy(acc_sh.at[pl.ds(base, ZC)],
                                out_hbm.at[cid, pl.ds(base, ZC)])

            return carry

        lax.fori_loop(0, ZPW, oc, 0)

    return scatter


_gather_h1 = _make_gather(0, E_H1)
_gather_h2 = _make_gather(E_H1, E_H2)
_scatter_h1 = _make_scatter(0, E_H1, 80)
_scatter_h2 = _make_scatter(E_H1, E_H2, 40)


def _ln(y, g, bt):
    mu = jnp.mean(y, axis=-1, keepdims=True)
    var = jnp.mean((y - mu) * (y - mu), axis=-1, keepdims=True)
    return (y - mu) * lax.rsqrt(var + 1e-5) * g + bt


def _embed_body(x_ref, w0, b0, w1, b1, g, bt, o_ref):
    h = jax.nn.silu(jnp.dot(x_ref[...], w0[...],
                            preferred_element_type=jnp.float32) + b0[...])
    y = jnp.dot(h, w1[...], preferred_element_type=jnp.float32) + b1[...]
    o_ref[...] = _ln(y, g[...], bt[...])


def _pq_body(n_ref, ws, wd, p_ref, q_ref):
    x = n_ref[...]
    p_ref[...] = jnp.dot(x, ws[...], preferred_element_type=jnp.float32)
    q_ref[...] = jnp.dot(x, wd[...], preferred_element_type=jnp.float32)


def _edge_body(gs_ref, gd_ref, e_ref, we, b0, w1, b1, g, bt, o_ref):
    x = e_ref[...]
    pre = gs_ref[...] + gd_ref[...] + jnp.dot(
        x, we[...], preferred_element_type=jnp.float32) + b0[...]
    h = jax.nn.silu(pre)
    y = jnp.dot(h, w1[...], preferred_element_type=jnp.float32) + b1[...]
    o_ref[...] = x + _ln(y, g[...], bt[...])


def _node_body(a_ref, c_ref, n_ref, wa, wn, b0, w1, b1, g, bt, o_ref):
    agg = a_ref[0] + a_ref[1] + c_ref[0] + c_ref[1]
    x = n_ref[...]
    pre = (jnp.dot(agg, wa[...], preferred_element_type=jnp.float32)
           + jnp.dot(x, wn[...], preferred_element_type=jnp.float32)
           + b0[...])
    h = jax.nn.silu(pre)
    y = jnp.dot(h, w1[...], preferred_element_type=jnp.float32) + b1[...]
    o_ref[...] = x + _ln(y, g[...], bt[...])


_N_TILE = 2000


def _etile(ecnt):
    for t in (2048, 1952, 1280, 640):
        if ecnt % t == 0:
            return t
    raise ValueError(ecnt)


def _full(shape):
    return pl.BlockSpec(shape, lambda i: (0,) * len(shape))


def _tc_embed(x, ecnt, w0, b0, w1, b1, g, bt):
    tile = _etile(ecnt)
    grid = (ecnt // tile,)
    return pl.pallas_call(
        _embed_body,
        grid=grid,
        in_specs=[
            pl.BlockSpec((tile, 4), lambda i: (i, 0)),
            _full((4, D)), _full((1, D)), _full((D, D)), _full((1, D)),
            _full((1, D)), _full((1, D)),
        ],
        out_specs=pl.BlockSpec((tile, D), lambda i: (i, 0)),
        out_shape=jax.ShapeDtypeStruct((ecnt, D), jnp.float32),
    )(x, w0, b0, w1, b1, g, bt)


def _tc_pq(nfeat, ws, wd):
    grid = (N_NODES // _N_TILE,)
    return pl.pallas_call(
        _pq_body,
        grid=grid,
        in_specs=[
            pl.BlockSpec((_N_TILE, D), lambda i: (i, 0)),
            _full((D, D)), _full((D, D)),
        ],
        out_specs=[pl.BlockSpec((_N_TILE, D), lambda i: (i, 0)),
                   pl.BlockSpec((_N_TILE, D), lambda i: (i, 0))],
        out_shape=[jax.ShapeDtypeStruct((N_NODES, D), jnp.float32),
                   jax.ShapeDtypeStruct((N_NODES, D), jnp.float32)],
    )(nfeat, ws, wd)


def _tc_edge(gs, gd, efeat, we, b0, w1, b1, g, bt):
    ecnt = gs.shape[0]
    tile = _etile(ecnt)
    grid = (ecnt // tile,)
    return pl.pallas_call(
        _edge_body,
        grid=grid,
        in_specs=[
            pl.BlockSpec((tile, D), lambda i: (i, 0)),
            pl.BlockSpec((tile, D), lambda i: (i, 0)),
            pl.BlockSpec((tile, D), lambda i: (i, 0)),
            _full((D, D)), _full((1, D)), _full((D, D)), _full((1, D)),
            _full((1, D)), _full((1, D)),
        ],
        out_specs=pl.BlockSpec((tile, D), lambda i: (i, 0)),
        out_shape=jax.ShapeDtypeStruct((ecnt, D), jnp.float32),
    )(gs, gd, efeat, we, b0, w1, b1, g, bt)


def _tc_node(a2, c2, nfeat, wa, wn, b0, w1, b1, g, bt):
    grid = (N_NODES // _N_TILE,)
    return pl.pallas_call(
        _node_body,
        grid=grid,
        in_specs=[
            pl.BlockSpec((_NC, _N_TILE, D), lambda i: (0, i, 0)),
            pl.BlockSpec((_NC, _N_TILE, D), lambda i: (0, i, 0)),
            pl.BlockSpec((_N_TILE, D), lambda i: (i, 0)),
            _full((D, D)), _full((D, D)), _full((1, D)), _full((D, D)),
            _full((1, D)), _full((1, D)), _full((1, D)),
        ],
        out_specs=pl.BlockSpec((_N_TILE, D), lambda i: (i, 0)),
        out_shape=jax.ShapeDtypeStruct((N_NODES, D), jnp.float32),
    )(a2, c2, nfeat, wa, wn, b0, w1, b1, g, bt)


def kernel(mesh_nfeat, edge_index, mesh_efeat,
           emb_W0, emb_b0, emb_W1, emb_b1, emb_g, emb_bt,
           We0, be0, We1, be1, eg, ebt,
           Wn0, bn0, Wn1, bn1, ng, nbt):
    r = lambda v: v.reshape(1, D)
    src = edge_index[0].astype(jnp.int32)
    dst = edge_index[1].astype(jnp.int32)

    ef1 = _tc_embed(mesh_efeat[:E_H1], E_H1, emb_W0, r(emb_b0), emb_W1,
                    r(emb_b1), r(emb_g), r(emb_bt))
    ef2 = _tc_embed(mesh_efeat[E_H1:], E_H2, emb_W0, r(emb_b0), emb_W1,
                    r(emb_b1), r(emb_g), r(emb_bt))
    nfeat = mesh_nfeat
    for i in range(N_BLOCK):
        p, q = _tc_pq(nfeat, We0[i, :D], We0[i, D:2 * D])
        gs1, gd1 = _gather_h1(p, q, src, dst)
        gs2, gd2 = _gather_h2(p, q, src, dst)
        ef1 = _tc_edge(gs1, gd1, ef1, We0[i, 2 * D:], r(be0[i]),
                       We1[i], r(be1[i]), r(eg[i]), r(ebt[i]))
        a2 = _scatter_h1(ef1, dst)
        ef2 = _tc_edge(gs2, gd2, ef2, We0[i, 2 * D:], r(be0[i]),
                       We1[i], r(be1[i]), r(eg[i]), r(ebt[i]))
        c2 = _scatter_h2(ef2, dst)
        nfeat = _tc_node(a2, c2, nfeat, Wn0[i, :D], Wn0[i, D:],
                         r(bn0[i]), Wn1[i], r(bn1[i]), r(ng[i]),
                         r(nbt[i]))
    return (nfeat, jnp.concatenate([ef1, ef2], axis=0))


# speedup vs baseline: 1.4134x; 1.0194x over previous
import functools

import jax
import jax.numpy as jnp
from jax import lax
from jax.experimental import pallas as pl
from jax.experimental.pallas import tpu as pltpu
from jax.experimental.pallas import tpu_sc as plsc

N_NODES = 10000
N_EDGES = 160000
D = 128
N_BLOCK = 4

_NC = 2
_NS = 16
_NW = _NC * _NS

E_H1 = 81920
E_H2 = N_EDGES - E_H1

ZC = 80
NZ = N_NODES // ZC
ZPW = -(-NZ // _NS)

_mesh = plsc.VectorSubcoreMesh(core_axis_name="c", subcore_axis_name="s")

GE = 80


def _make_gather(eoff, ecnt):
    ept = ecnt // _NS
    gn = ept // GE
    assert ept * _NS == ecnt and gn * GE == ept and ept % 8 == 0

    @functools.partial(
        pl.kernel,
        out_type=(jax.ShapeDtypeStruct((ecnt, D), jnp.float32),
                  jax.ShapeDtypeStruct((ecnt, D), jnp.float32)),
        mesh=_mesh,
        scratch_types=[
            pltpu.VMEM_SHARED((N_NODES, D), jnp.float32),
            pltpu.VMEM((GE,), jnp.int32),
            pltpu.VMEM((GE,), jnp.int32),
            pltpu.VMEM((GE, D), jnp.float32),
            pltpu.VMEM((GE, D), jnp.float32),
            pltpu.SemaphoreType.DMA,
            pltpu.SemaphoreType.DMA,
            pltpu.SemaphoreType.DMA,
            pltpu.SemaphoreType.DMA,
            pltpu.SemaphoreType.DMA,
            pltpu.SemaphoreType.DMA,
        ],
    )
    def gather(p_hbm, q_hbm, src_hbm, dst_hbm, gs_hbm, gd_hbm,
               tbl_sh, idx0, idx1, buf0, buf1,
               isem0, isem1, gsem0, gsem1, wsem0, wsem1):
        cid = lax.axis_index("c")
        sid = lax.axis_index("s")
        idxs = (idx0, idx1)
        bufs = (buf0, buf1)
        isems = (isem0, isem1)
        gsems = (gsem0, gsem1)
        wsems = (wsem0, wsem1)

        def stage(tab_hbm):
            def st(z, carry):
                ci = z * _NS + sid

                @pl.when(ci < NZ)
                def _():
                    base = pl.multiple_of(ci * ZC, 8)
                    pltpu.sync_copy(tab_hbm.at[pl.ds(base, ZC)],
                                    tbl_sh.at[pl.ds(base, ZC)])

                return carry

            lax.fori_loop(0, ZPW, st, 0)

        @pl.when(cid == 0)
        def _():
            stage(p_hbm)

        @pl.when(cid == 1)
        def _():
            stage(q_hbm)

        plsc.subcore_barrier()

        def run(ind_hbm, out_hbm):
            tb_i = eoff + sid * ept
            tb_o = sid * ept

            def idxload(k, b):
                pltpu.async_copy(ind_hbm.at[pl.ds(tb_i + k * GE, GE)],
                                 idxs[b], isems[b])

            idxload(0, 0)
            idxload(1, 1)
            pltpu.make_async_copy(ind_hbm.at[pl.ds(0, GE)], idx0,
                                  isem0).wait()
            pltpu.async_copy(tbl_sh.at[idx0], buf0, gsem0)

            def step(k, b):
                o = 1 - b
                pltpu.make_async_copy(tbl_sh.at[idxs[b]], bufs[b],
                                      gsems[b]).wait()

                @pl.when(k + 2 < gn)
                def _():
                    idxload(k + 2, b)

                pltpu.async_copy(bufs[b],
                                 out_hbm.at[pl.ds(tb_o + k * GE, GE)],
                                 wsems[b])
                pltpu.make_async_copy(ind_hbm.at[pl.ds(0, GE)], idxs[o],
                                      isems[o]).wait()

                @pl.when(k >= 1)
                def _():
                    pltpu.make_async_copy(bufs[o],
                                          out_hbm.at[pl.ds(0, GE)],
                                          wsems[o]).wait()

                pltpu.async_copy(tbl_sh.at[idxs[o]], bufs[o], gsems[o])

            def body(k2, carry):
                step(k2 * 2, 0)
                step(k2 * 2 + 1, 1)
                return carry

            lax.fori_loop(0, (gn - 1) // 2, body, 0)

            if gn % 2 == 1:
                pltpu.make_async_copy(tbl_sh.at[idx0], buf0,
                                      gsem0).wait()
                pltpu.async_copy(
                    buf0, out_hbm.at[pl.ds(tb_o + (gn - 1) * GE, GE)],
                    wsem0)
                pltpu.make_async_copy(buf1, out_hbm.at[pl.ds(0, GE)],
                                      wsem1).wait()
                pltpu.make_async_copy(buf0, out_hbm.at[pl.ds(0, GE)],
                                      wsem0).wait()
            else:
                pltpu.make_async_copy(tbl_sh.at[idx0], buf0,
                                      gsem0).wait()
                pltpu.async_copy(
                    buf0, out_hbm.at[pl.ds(tb_o + (gn - 2) * GE, GE)],
                    wsem0)
                pltpu.make_async_copy(ind_hbm.at[pl.ds(0, GE)], idx1,
                                      isem1).wait()
                pltpu.make_async_copy(buf1, out_hbm.at[pl.ds(0, GE)],
                                      wsem1).wait()
                pltpu.async_copy(tbl_sh.at[idx1], buf1, gsem1)
                pltpu.make_async_copy(tbl_sh.at[idx1], buf1,
                                      gsem1).wait()
                pltpu.async_copy(
                    buf1, out_hbm.at[pl.ds(tb_o + (gn - 1) * GE, GE)],
                    wsem1)
                pltpu.make_async_copy(buf0, out_hbm.at[pl.ds(0, GE)],
                                      wsem0).wait()
                pltpu.make_async_copy(buf1, out_hbm.at[pl.ds(0, GE)],
                                      wsem1).wait()

        @pl.when(cid == 0)
        def _():
            run(src_hbm, gs_hbm)

        @pl.when(cid == 1)
        def _():
            run(dst_hbm, gd_hbm)

    return gather


def _make_scatter(eoff, ecnt, se):
    epw = ecnt // _NW
    sn = epw // se
    assert epw * _NW == ecnt and sn * se == epw and epw % 8 == 0
    assert sn >= 2 and se % 8 == 0

    @functools.partial(
        pl.kernel,
        out_type=jax.ShapeDtypeStruct((_NC, N_NODES, D), jnp.float32),
        mesh=_mesh,
        scratch_types=[
            pltpu.VMEM_SHARED((N_NODES, D), jnp.float32),
            pltpu.VMEM((se, D), jnp.float32),
            pltpu.VMEM((se, D), jnp.float32),
            pltpu.VMEM((se,), jnp.int32),
            pltpu.VMEM((se,), jnp.int32),
            pltpu.SemaphoreType.DMA,
            pltpu.SemaphoreType.DMA,
            pltpu.SemaphoreType.DMA,
            pltpu.SemaphoreType.DMA,
            pltpu.SemaphoreType.DMA,
            pltpu.SemaphoreType.DMA,
        ],
    )
    def scatter(e_hbm, dst_hbm, out_hbm, acc_sh, ebuf0, ebuf1,
                idx0, idx1,
                esem0, esem1, isem0, isem1, ssem0, ssem1):
        cid = lax.axis_index("c")
        sid = lax.axis_index("s")
        wid = sid * _NC + cid
        wb_e = wid * epw
        wb_d = eoff + wid * epw
        ebufs = (ebuf0, ebuf1)
        idxs = (idx0, idx1)
        esems = (esem0, esem1)
        isems = (isem0, isem1)
        ssems = (ssem0, ssem1)

        def zb(i, carry):
            r = i // (D // 16)
            c2 = (i % (D // 16)) * 16
            ebuf0[r, pl.ds(c2, 16)] = jnp.zeros((16,), jnp.float32)
            return carry

        lax.fori_loop(0, se * (D // 16), zb, 0)
        zrep = -(-ZC // se)

        def zc(z, carry):
            ci = z * _NS + sid

            @pl.when(ci < NZ)
            def _():
                for j in range(zrep):
                    base = pl.multiple_of(ci * ZC + j * se, 8)
                    pltpu.sync_copy(ebuf0,
                                    acc_sh.at[pl.ds(base, se)])

                return None

            return carry

        lax.fori_loop(0, ZPW, zc, 0)
        plsc.subcore_barrier()

        def load(k, b):
            pltpu.async_copy(e_hbm.at[pl.ds(wb_e + k * se, se)],
                             ebufs[b], esems[b])
            pltpu.async_copy(dst_hbm.at[pl.ds(wb_d + k * se, se)],
                             idxs[b], isems[b])

        load(0, 0)

        def step(k, b):
            o = 1 - b
            pltpu.make_async_copy(e_hbm.at[pl.ds(0, se)], ebufs[b],
                                  esems[b]).wait()
            pltpu.make_async_copy(dst_hbm.at[pl.ds(0, se)], idxs[b],
                                  isems[b]).wait()
            pltpu.async_copy(ebufs[b], acc_sh.at[idxs[b]], ssems[b],
                             add=True)

            @pl.when(k >= 1)
            def _():
                pltpu.make_async_copy(ebufs[o], acc_sh.at[idxs[o]],
                                      ssems[o]).wait()

            @pl.when(k + 1 < sn)
            def _():
                load(k + 1, o)

        def body(k2, carry):
            step(k2 * 2, 0)
            step(k2 * 2 + 1, 1)
            return carry

        lax.fori_loop(0, sn // 2, body, 0)

        if sn % 2 == 1:
            pltpu.make_async_copy(e_hbm.at[pl.ds(0, se)], ebuf0,
                                  esem0).wait()
            pltpu.make_async_copy(dst_hbm.at[pl.ds(0, se)], idx0,
                                  isem0).wait()
            pltpu.async_copy(ebuf0, acc_sh.at[idx0], ssem0, add=True)
            pltpu.make_async_copy(ebuf1, acc_sh.at[idx1], ssem1).wait()
            pltpu.make_async_copy(ebuf0, acc_sh.at[idx0], ssem0).wait()
        else:
            pltpu.make_async_copy(ebuf1, acc_sh.at[idx1], ssem1).wait()
        plsc.subcore_barrier()

        def oc(z, carry):
            ci = z * _NS + sid

            @pl.when(ci < NZ)
            def _():
                base = pl.multiple_of(ci * ZC, 8)
                pltpu.sync_copy(acc_sh.at[pl.ds(base, ZC)],
                                out_hbm.at[cid, pl.ds(base, ZC)])

            return carry

        lax.fori_loop(0, ZPW, oc, 0)

    return scatter


_gather_h1 = _make_gather(0, E_H1)
_gather_h2 = _make_gather(E_H1, E_H2)
_scatter_h1 = _make_scatter(0, E_H1, 80)
_scatter_h2 = _make_scatter(E_H1, E_H2, 40)


def _ln(y, g, bt):
    mu = jnp.mean(y, axis=-1, keepdims=True)
    var = jnp.mean((y - mu) * (y - mu), axis=-1, keepdims=True)
    return (y - mu) * lax.rsqrt(var + 1e-5) * g + bt


def _embed_body(x_ref, w0, b0, w1, b1, g, bt, o_ref):
    h = jax.nn.silu(jnp.dot(x_ref[...], w0[...],
                            preferred_element_type=jnp.float32) + b0[...])
    y = jnp.dot(h, w1[...], preferred_element_type=jnp.float32) + b1[...]
    o_ref[...] = _ln(y, g[...], bt[...])


def _pq_body(n_ref, ws, wd, p_ref, q_ref):
    x = n_ref[...]
    p_ref[...] = jnp.dot(x, ws[...], preferred_element_type=jnp.float32)
    q_ref[...] = jnp.dot(x, wd[...], preferred_element_type=jnp.float32)


def _edge_body(gs_ref, gd_ref, e_ref, we, b0, w1, b1, g, bt, o_ref):
    x = e_ref[...]
    pre = gs_ref[...] + gd_ref[...] + jnp.dot(
        x, we[...], preferred_element_type=jnp.float32) + b0[...]
    h = jax.nn.silu(pre)
    y = jnp.dot(h, w1[...], preferred_element_type=jnp.float32) + b1[...]
    o_ref[...] = x + _ln(y, g[...], bt[...])


def _node_body(a_ref, c_ref, n_ref, wa, wn, b0, w1, b1, g, bt, o_ref):
    agg = a_ref[0] + a_ref[1] + c_ref[0] + c_ref[1]
    x = n_ref[...]
    pre = (jnp.dot(agg, wa[...], preferred_element_type=jnp.float32)
           + jnp.dot(x, wn[...], preferred_element_type=jnp.float32)
           + b0[...])
    h = jax.nn.silu(pre)
    y = jnp.dot(h, w1[...], preferred_element_type=jnp.float32) + b1[...]
    o_ref[...] = x + _ln(y, g[...], bt[...])


_N_TILE = 2000


def _etile(ecnt):
    for t in (4096, 3904, 2048, 1952, 1280, 640):
        if ecnt % t == 0:
            return t
    raise ValueError(ecnt)


def _full(shape):
    return pl.BlockSpec(shape, lambda i: (0,) * len(shape))


def _tc_embed(x, ecnt, w0, b0, w1, b1, g, bt):
    tile = _etile(ecnt)
    grid = (ecnt // tile,)
    return pl.pallas_call(
        _embed_body,
        grid=grid,
        in_specs=[
            pl.BlockSpec((tile, 4), lambda i: (i, 0)),
            _full((4, D)), _full((1, D)), _full((D, D)), _full((1, D)),
            _full((1, D)), _full((1, D)),
        ],
        out_specs=pl.BlockSpec((tile, D), lambda i: (i, 0)),
        out_shape=jax.ShapeDtypeStruct((ecnt, D), jnp.float32),
    )(x, w0, b0, w1, b1, g, bt)


def _tc_pq(nfeat, ws, wd):
    grid = (N_NODES // _N_TILE,)
    return pl.pallas_call(
        _pq_body,
        grid=grid,
        in_specs=[
            pl.BlockSpec((_N_TILE, D), lambda i: (i, 0)),
            _full((D, D)), _full((D, D)),
        ],
        out_specs=[pl.BlockSpec((_N_TILE, D), lambda i: (i, 0)),
                   pl.BlockSpec((_N_TILE, D), lambda i: (i, 0))],
        out_shape=[jax.ShapeDtypeStruct((N_NODES, D), jnp.float32),
                   jax.ShapeDtypeStruct((N_NODES, D), jnp.float32)],
    )(nfeat, ws, wd)


def _tc_edge(gs, gd, efeat, we, b0, w1, b1, g, bt):
    ecnt = gs.shape[0]
    tile = _etile(ecnt)
    grid = (ecnt // tile,)
    return pl.pallas_call(
        _edge_body,
        grid=grid,
        in_specs=[
            pl.BlockSpec((tile, D), lambda i: (i, 0)),
            pl.BlockSpec((tile, D), lambda i: (i, 0)),
            pl.BlockSpec((tile, D), lambda i: (i, 0)),
            _full((D, D)), _full((1, D)), _full((D, D)), _full((1, D)),
            _full((1, D)), _full((1, D)),
        ],
        out_specs=pl.BlockSpec((tile, D), lambda i: (i, 0)),
        out_shape=jax.ShapeDtypeStruct((ecnt, D), jnp.float32),
    )(gs, gd, efeat, we, b0, w1, b1, g, bt)


def _tc_node(a2, c2, nfeat, wa, wn, b0, w1, b1, g, bt):
    grid = (N_NODES // _N_TILE,)
    return pl.pallas_call(
        _node_body,
        grid=grid,
        in_specs=[
            pl.BlockSpec((_NC, _N_TILE, D), lambda i: (0, i, 0)),
            pl.BlockSpec((_NC, _N_TILE, D), lambda i: (0, i, 0)),
            pl.BlockSpec((_N_TILE, D), lambda i: (i, 0)),
            _full((D, D)), _full((D, D)), _full((1, D)), _full((D, D)),
            _full((1, D)), _full((1, D)), _full((1, D)),
        ],
        out_specs=pl.BlockSpec((_N_TILE, D), lambda i: (i, 0)),
        out_shape=jax.ShapeDtypeStruct((N_NODES, D), jnp.float32),
    )(a2, c2, nfeat, wa, wn, b0, w1, b1, g, bt)


def kernel(mesh_nfeat, edge_index, mesh_efeat,
           emb_W0, emb_b0, emb_W1, emb_b1, emb_g, emb_bt,
           We0, be0, We1, be1, eg, ebt,
           Wn0, bn0, Wn1, bn1, ng, nbt):
    r = lambda v: v.reshape(1, D)
    src = edge_index[0].astype(jnp.int32)
    dst = edge_index[1].astype(jnp.int32)

    ef1 = _tc_embed(mesh_efeat[:E_H1], E_H1, emb_W0, r(emb_b0), emb_W1,
                    r(emb_b1), r(emb_g), r(emb_bt))
    ef2 = _tc_embed(mesh_efeat[E_H1:], E_H2, emb_W0, r(emb_b0), emb_W1,
                    r(emb_b1), r(emb_g), r(emb_bt))
    nfeat = mesh_nfeat
    for i in range(N_BLOCK):
        p, q = _tc_pq(nfeat, We0[i, :D], We0[i, D:2 * D])
        gs1, gd1 = _gather_h1(p, q, src, dst)
        gs2, gd2 = _gather_h2(p, q, src, dst)
        ef1 = _tc_edge(gs1, gd1, ef1, We0[i, 2 * D:], r(be0[i]),
                       We1[i], r(be1[i]), r(eg[i]), r(ebt[i]))
        a2 = _scatter_h1(ef1, dst)
        ef2 = _tc_edge(gs2, gd2, ef2, We0[i, 2 * D:], r(be0[i]),
                       We1[i], r(be1[i]), r(eg[i]), r(ebt[i]))
        c2 = _scatter_h2(ef2, dst)
        nfeat = _tc_node(a2, c2, nfeat, Wn0[i, :D], Wn0[i, D:],
                         r(bn0[i]), Wn1[i], r(bn1[i]), r(ng[i]),
                         r(nbt[i]))
    return (nfeat, jnp.concatenate([ef1, ef2], axis=0))
